# Initial kernel scaffold; baseline (speedup 1.0000x reference)
import jax
import jax.numpy as jnp
from jax.experimental import pallas as pl


def kernel(key_phi, key_embed):
    raise NotImplementedError("write your pallas kernel here")


# speedup vs baseline: 2.3310x; 2.3310x over previous
import numpy as np

import jax
import jax.numpy as jnp
from jax import lax
from jax.experimental import pallas as pl
from jax.experimental.pallas import tpu as pltpu
from jax.experimental.pallas import tpu_sc as plsc

_N = 262144
_C = 128
_R = _N // _C
_D = 64
_LOGN = 18

_NC = 2
_NS = 16
_NW = _NC * _NS
_CH = 128


def _stage_table():
    rows = []
    for k in range(1, _LOGN + 1):
        j = 1 << (k - 1)
        while j >= 1:
            if j >= _C:
                d = j // _C
                axis, dist, ps = 1, d, d.bit_length() - 1
            else:
                axis, dist, ps = 0, j, j.bit_length() - 1
            da, ds = (0, k) if k <= 6 else (1, k - 7)
            rows.append((axis, dist, ps, da, ds))
            j >>= 1
    return np.asarray(rows, dtype=np.int32)


_STAGES = _stage_table()
_NSTAGES = len(_STAGES)


def _lex_gt(ap, ai, bp, bi):
    return (ap > bp) | ((ap == bp) & (ai > bi))


def _sort_body(params_ref, phi_ref, sorted_ref, idx_ref):
    r_io = lax.broadcasted_iota(jnp.int32, (_R, _C), 0)
    c_io = lax.broadcasted_iota(jnp.int32, (_R, _C), 1)

    def make_branch(axis):
        size = (_C, _R)[axis]
        dim = 1 - axis
        pos = (c_io, r_io)[axis]

        def branch(phi, idx, dist, ps, da, ds):
            is_b = ((pos >> ps) & 1) == 1
            fwd_p = pltpu.roll(phi, dist, dim)
            bwd_p = pltpu.roll(phi, size - dist, dim)
            fwd_i = pltpu.roll(idx, dist, dim)
            bwd_i = pltpu.roll(idx, size - dist, dim)
            pp = jnp.where(is_b, fwd_p, bwd_p)
            pi = jnp.where(is_b, fwd_i, bwd_i)
            dio = jnp.where(da == 0, c_io, r_io)
            down = ((dio >> ds) & 1) == 1
            gt = _lex_gt(phi, idx, pp, pi)
            take = gt ^ down ^ is_b
            return jnp.where(take, pp, phi), jnp.where(take, pi, idx)

        return branch

    lane_b = make_branch(0)
    row_b = make_branch(1)

    def step(t, carry):
        phi, idx = carry
        axis = params_ref[t, 0]
        dist = params_ref[t, 1]
        ps = params_ref[t, 2]
        da = params_ref[t, 3]
        ds = params_ref[t, 4]
        return lax.switch(axis, (lane_b, row_b), phi, idx, dist, ps, da, ds)

    phi0 = phi_ref[...]
    idx0 = r_io * _C + c_io
    phi, idx = lax.fori_loop(0, _NSTAGES, step, (phi0, idx0))
    sorted_ref[...] = phi
    idx_ref[...] = idx


def _sort(phi2):
    return pl.pallas_call(
        _sort_body,
        in_specs=[
            pl.BlockSpec(memory_space=pltpu.SMEM),
            pl.BlockSpec(memory_space=pltpu.VMEM),
        ],
        out_shape=[
            jax.ShapeDtypeStruct((_R, _C), jnp.float32),
            jax.ShapeDtypeStruct((_R, _C), jnp.int32),
        ],
    )(jnp.asarray(_STAGES), phi2)


def _gather_body(emb_hbm, idx_hbm, out_hbm, idx_v, rows_v, sem):
    wid = lax.axis_index("s") * _NC + lax.axis_index("c")
    n_chunks = _N // (_NW * _CH)
    pltpu.sync_copy(idx_hbm.at[pl.ds(wid * n_chunks, n_chunks)], idx_v)

    def step(q, carry):
        pltpu.async_copy(emb_hbm.at[idx_v.at[q]], rows_v, sem).wait()
        row0 = (wid * n_chunks + q) * _CH
        pltpu.sync_copy(rows_v, out_hbm.at[pl.ds(row0, _CH)])
        return carry

    lax.fori_loop(0, n_chunks, step, 0)


def _gather(emb, idx2):
    n_chunks = _N // (_NW * _CH)
    mesh = plsc.VectorSubcoreMesh(core_axis_name="c", subcore_axis_name="s")
    f = pl.kernel(
        _gather_body,
        out_type=jax.ShapeDtypeStruct((_N, _D), jnp.float32),
        mesh=mesh,
        compiler_params=pltpu.CompilerParams(use_tc_tiling_on_sc=False),
        scratch_types=[
            pltpu.VMEM((n_chunks, _CH), jnp.int32),
            pltpu.VMEM((_CH, _D), jnp.float32),
            pltpu.SemaphoreType.DMA,
        ],
    )
    return f(emb, idx2)


def kernel(key_phi, key_embed):
    phi2 = key_phi.reshape(_R, _C)
    sorted_phi, idx2 = _sort(phi2)
    emb = key_embed.reshape(_N, _D)
    out = _gather(emb, idx2)
    return (sorted_phi.reshape(1, _N), out.reshape(1, _N, _D))
